# baseline (speedup 1.0000x reference)
import functools

import jax
import jax.numpy as jnp
from jax import lax
from jax.experimental import pallas as pl
from jax.experimental.pallas import tpu as pltpu
from jax.experimental.pallas import tpu_sc as plsc

N = 100000
E = 1600000
G = 64
IN_DIM = 24
GLOBAL_DIM = 10
H = 64
HEADS = 4
C = 16

NSC = 2
NTC = 16
SL = 4

K = 1024
GCH = K // 128
N_PAD = 100352
E_PAD = 98 * NTC * K
PER_TILE = E_PAD // NTC
N_CHUNKS = PER_TILE // K
ZROWS = N_PAD // NTC
ZB = 448

_mesh = plsc.VectorSubcoreMesh(core_axis_name="c", subcore_axis_name="s")


@functools.partial(
    pl.kernel,
    out_type=jax.ShapeDtypeStruct((SL, N_PAD, 16), jnp.float32),
    mesh=_mesh,
    scratch_types=[
        pltpu.VMEM((GCH, 128), jnp.int32),
        pltpu.VMEM((GCH, 128), jnp.int32),
        pltpu.VMEM((K, 16), jnp.float32),
        pltpu.VMEM((ZB, 16), jnp.float32),
        pltpu.VMEM_SHARED((N_PAD, 16), jnp.float32),
        pltpu.SemaphoreType.DMA,
    ],
    compiler_params=pltpu.CompilerParams(use_tc_tiling_on_sc=False),
    name="gin_agg",
)
def _gin_agg(hsl, src2, dst2, out, sidx, didx, rows, zbuf, acc, sem):
    c = lax.axis_index("c")
    s = lax.axis_index("s")

    @pl.loop(0, ZB)
    def _zero_zbuf(i):
        zbuf[i] = jnp.zeros((16,), jnp.float32)

    for si in range(2):
        sl_id = 2 * c + si
        hsl_s = hsl.at[sl_id]
        for j in range(ZROWS // ZB):
            pltpu.sync_copy(zbuf, acc.at[pl.ds(s * ZROWS + j * ZB, ZB)])
        plsc.subcore_barrier()

        @pl.loop(0, N_CHUNKS)
        def _chunk(ci):
            rb = s * (PER_TILE // 128) + ci * GCH
            pltpu.sync_copy(src2.at[pl.ds(rb, GCH)], sidx)
            pltpu.sync_copy(dst2.at[pl.ds(rb, GCH)], didx)
            descs = []
            for g in range(GCH):
                descs.append(pltpu.async_copy(
                    hsl_s.at[sidx.at[g]], rows.at[pl.ds(g * 128, 128)], sem))
            for d in descs:
                d.wait()
            for g in range(GCH):
                pltpu.sync_copy(rows.at[pl.ds(g * 128, 128)],
                                acc.at[didx.at[g]], add=True)

        plsc.subcore_barrier()
        pltpu.sync_copy(acc.at[pl.ds(s * ZROWS, ZROWS)],
                        out.at[sl_id, pl.ds(s * ZROWS, ZROWS)])
        plsc.subcore_barrier()


PTA = E_PAD // (NSC * NTC)
N_CH_A = PTA // K


@functools.partial(
    pl.kernel,
    out_type=(
        jax.ShapeDtypeStruct((E_PAD // 2, 16), jnp.float32),
        jax.ShapeDtypeStruct((NSC, N_PAD, 16), jnp.float32),
    ),
    mesh=_mesh,
    scratch_types=[
        pltpu.VMEM((GCH, 128), jnp.int32),
        pltpu.VMEM((GCH, 128), jnp.int32),
        pltpu.VMEM((K, 16), jnp.float32),
        pltpu.VMEM((K, 16), jnp.float32),
        pltpu.VMEM((K // 2, 16), jnp.float32),
        pltpu.VMEM((ZB, 16), jnp.float32),
        pltpu.VMEM_SHARED((N_PAD // 2 + 8, 16), jnp.float32),
        pltpu.SemaphoreType.DMA,
    ],
    compiler_params=pltpu.CompilerParams(use_tc_tiling_on_sc=False),
    name="gat_att",
)
def _gat_att(asrc16, adst16, src2, dst2, ex4, den, sidx, didx, abuf, bbuf,
             pbuf, zbuf, acc, sem):
    c = lax.axis_index("c")
    s = lax.axis_index("s")
    w = s * NSC + c
    HN = N_PAD // 2

    @pl.loop(0, ZB)
    def _zz(i):
        zbuf[i] = jnp.zeros((16,), jnp.float32)

    iot = lax.iota(jnp.int32, 16)
    rot = (iot - 8) & 15
    unrot = (iot + 8) & 15
    msk = iot < 8

    for half in range(2):
        for j in range(ZROWS // ZB // 2):
            pltpu.sync_copy(zbuf, acc.at[pl.ds(s * (ZROWS // 2) + j * ZB, ZB)])
        pltpu.sync_copy(zbuf.at[pl.ds(0, 8)], acc.at[pl.ds(HN, 8)])
        plsc.subcore_barrier()

        @pl.loop(0, N_CH_A)
        def _chunk(ci):
            rb = w * (PTA // 128) + ci * GCH
            pltpu.sync_copy(dst2.at[pl.ds(rb, GCH)], didx)
            pltpu.sync_copy(src2.at[pl.ds(rb, GCH)], sidx)
            descs = []
            for g in range(GCH):
                descs.append(pltpu.async_copy(
                    asrc16.at[sidx.at[g]], abuf.at[pl.ds(g * 128, 128)],
                    sem))
                descs.append(pltpu.async_copy(
                    adst16.at[didx.at[g]], bbuf.at[pl.ds(g * 128, 128)],
                    sem))

            for g in range(GCH):
                @pl.loop(0, 8)
                def _remap(j):
                    v = didx[g, pl.ds(j * 16, 16)] - half * HN
                    ok = (v >= 0) & (v < HN)
                    didx[g, pl.ds(j * 16, 16)] = jnp.where(ok, v, HN)

            for d in descs:
                d.wait()

            @pl.loop(0, K, unroll=8)
            def _edge(j):
                v = abuf[j] + bbuf[j]
                v = jnp.where(v > 0, v, 0.2 * v)
                abuf[j] = jnp.exp(v)

            if half == 0:
                @pl.loop(0, K // 2, unroll=8)
                def _pair(q):
                    a = abuf[2 * q]
                    b = abuf[2 * q + 1]
                    pbuf[q] = jnp.where(msk, a, b[rot])

                pltpu.sync_copy(
                    pbuf, ex4.at[pl.ds((w * PTA + ci * K) // 2, K // 2)])

            for g in range(GCH):
                pltpu.sync_copy(abuf.at[pl.ds(g * 128, 128)],
                                acc.at[didx.at[g]], add=True)

        plsc.subcore_barrier()
        pltpu.sync_copy(
            acc.at[pl.ds(s * (ZROWS // 2), ZROWS // 2)],
            den.at[c, pl.ds(half * HN + s * (ZROWS // 2), ZROWS // 2)])
        plsc.subcore_barrier()


@functools.partial(
    pl.kernel,
    out_type=jax.ShapeDtypeStruct((HEADS, N_PAD, 16), jnp.float32),
    mesh=_mesh,
    scratch_types=[
        pltpu.VMEM((2, GCH, 128), jnp.int32),
        pltpu.VMEM((2, GCH, 128), jnp.int32),
        pltpu.VMEM((2, K, 16), jnp.float32),
        pltpu.VMEM((2, K // 2, 16), jnp.float32),
        pltpu.VMEM((ZB, 16), jnp.float32),
        pltpu.VMEM_SHARED((N_PAD // 2 + 8, 16), jnp.float32),
        pltpu.SemaphoreType.DMA,
        pltpu.SemaphoreType.DMA,
    ],
    compiler_params=pltpu.CompilerParams(use_tc_tiling_on_sc=False),
    name="gat_agg",
)
def _gat_agg(xh_sl, ex4, src2, dst2, out, sidx, didx, rows, exv, zbuf,
             acc, gsem, ssem):
    c = lax.axis_index("c")
    s = lax.axis_index("s")
    HN = N_PAD // 2

    @pl.loop(0, ZB)
    def _zz(i):
        zbuf[i] = jnp.zeros((16,), jnp.float32)

    def fire(ci, b, xh_h):
        base = s * PER_TILE + ci * K
        rb = base // 128
        base2 = s * (PER_TILE // 2) + ci * (K // 2)
        pltpu.sync_copy(src2.at[pl.ds(rb, GCH)], sidx.at[b])
        pltpu.sync_copy(dst2.at[pl.ds(rb, GCH)], didx.at[b])
        pltpu.sync_copy(ex4.at[pl.ds(base2, K // 2)], exv.at[b])

        @pl.loop(0, GCH)
        def _fire(g):
            pltpu.async_copy(
                xh_h.at[sidx.at[b].at[g]],
                rows.at[b].at[pl.ds(g * 128, 128)], gsem)

    def process(b, hd, half, xh_h):
        pltpu.make_async_copy(
            xh_h.at[pl.ds(0, K)], rows.at[b], gsem).wait()
        for g in range(GCH):
            @pl.loop(0, 8)
            def _remap(j):
                v = didx[b, g, pl.ds(j * 16, 16)] - half * HN
                ok = (v >= 0) & (v < HN)
                didx[b, g, pl.ds(j * 16, 16)] = jnp.where(ok, v, HN)

        lane0 = jnp.full((16,), hd, jnp.int32)
        lane1 = jnp.full((16,), 8 + hd, jnp.int32)

        @pl.loop(0, K // 2, unroll=8)
        def _scale(q):
            v = exv[b, q]
            rows[b, 2 * q] = rows[b, 2 * q] * v[lane0]
            rows[b, 2 * q + 1] = rows[b, 2 * q + 1] * v[lane1]

        @pl.loop(0, GCH)
        def _scat(g):
            pltpu.async_copy(rows.at[b].at[pl.ds(g * 128, 128)],
                             acc.at[didx.at[b].at[g]], ssem, add=True)

    def drain_scatter(b):
        pltpu.make_async_copy(acc.at[pl.ds(0, K)], rows.at[b], ssem).wait()

    for si in range(2):
        hd = 2 * c + si
        xh_h = xh_sl.at[hd]
        for half in range(2):
            for j in range(ZROWS // ZB // 2):
                pltpu.sync_copy(
                    zbuf, acc.at[pl.ds(s * (ZROWS // 2) + j * ZB, ZB)])
            pltpu.sync_copy(zbuf.at[pl.ds(0, 8)], acc.at[pl.ds(HN, 8)])
            plsc.subcore_barrier()

            fire(0, 0, xh_h)

            @pl.loop(0, N_CHUNKS - 1)
            def _chunk(ci):
                b = ci % 2
                fire(ci + 1, 1 - b, xh_h)
                process(b, hd, half, xh_h)
                drain_scatter(b)

            process((N_CHUNKS - 1) % 2, hd, half, xh_h)
            drain_scatter((N_CHUNKS - 1) % 2)

            plsc.subcore_barrier()
            pltpu.sync_copy(
                acc.at[pl.ds(s * (ZROWS // 2), ZROWS // 2)],
                out.at[hd, pl.ds(half * HN + s * (ZROWS // 2), ZROWS // 2)])
            plsc.subcore_barrier()


def _segment_sum_edges(h, src2, dst2):
    hp = jnp.pad(h, ((0, N_PAD - N), (0, 0)))
    hsl = hp.reshape(N_PAD, SL, 16).transpose(1, 0, 2)
    agg_sl = _gin_agg(hsl, src2, dst2)
    return agg_sl.transpose(1, 0, 2).reshape(N_PAD, H)[:N]


def _ln(x, g, b):
    mu = jnp.mean(x, axis=-1, keepdims=True)
    var = jnp.var(x, axis=-1, keepdims=True)
    return (x - mu) / jnp.sqrt(var + 1e-5) * g + b


def _lin(x, W, b):
    return x @ W + b


def _gin_conv(x, src2, dst2, p):
    agg = _segment_sum_edges(x, src2, dst2)
    h = (1.0 + p['eps']) * x + agg
    h = _lin(h, p['W1'], p['b1'])
    h = h / jnp.sqrt(1.0 + 1e-5) * p['bn_g'] + p['bn_b']
    h = jax.nn.relu(h)
    return _lin(h, p['W2'], p['b2'])


def _gat_conv(x, src2, dst2, p):
    xh = x @ p['W']
    xhr = xh.reshape(N, HEADS, C)
    a_src = jnp.sum(xhr * p['att_src'], axis=-1)
    a_dst = jnp.sum(xhr * p['att_dst'], axis=-1)
    asrc16 = jnp.pad(a_src, ((0, N_PAD - N), (0, 16 - HEADS)))
    adst16 = jnp.pad(a_dst, ((0, N_PAD - N), (0, 16 - HEADS)))
    ex4, den2 = _gat_att(asrc16, adst16, src2, dst2)
    exl = jnp.exp(jax.nn.leaky_relu(a_src + a_dst, 0.2))
    den = den2[0, :N, :HEADS] + den2[1, :N, :HEADS] + exl
    xh_sl = jnp.pad(xh, ((0, N_PAD - N), (0, 0))).reshape(
        N_PAD, HEADS, C).transpose(1, 0, 2)
    agg_sl = _gat_agg(xh_sl, ex4, src2, dst2)
    agg = agg_sl.transpose(1, 0, 2)[:N]
    outr = (agg + exl[:, :, None] * xhr) / (den[:, :, None] + 1e-16)
    return outr.reshape(N, HEADS * C) + p['bias']


def kernel(x, edge_index, batch, global_feats, params):
    src = edge_index[0]
    dst = edge_index[1]
    pad = jnp.full((E_PAD - E,), N, jnp.int32)
    src2 = jnp.concatenate([src, pad]).reshape(E_PAD // 128, 128)
    dst2 = jnp.concatenate([dst, pad]).reshape(E_PAD // 128, 128)

    h = jnp.concatenate([x, global_feats[batch]], axis=1)
    pr = params['proj']
    h = jax.nn.relu(_ln(_lin(h, pr['W'], pr['b']), pr['ln_g'], pr['ln_b']))
    for p in params['gin']:
        idn = h
        h = jax.nn.relu(_ln(_gin_conv(h, src2, dst2, p), p['ln_g'], p['ln_b'])) + idn
    gin_out = h
    for p in params['gat']:
        idn = h
        h = jax.nn.elu(_ln(_gat_conv(h, src2, dst2, p), p['ln_g'], p['ln_b'])) + idn
    gat_out = h
    add_p = jax.ops.segment_sum(gin_out, batch, num_segments=G)
    cnt = jax.ops.segment_sum(jnp.ones((N,), dtype=jnp.float32), batch, num_segments=G)
    mean_p = jax.ops.segment_sum(gat_out, batch, num_segments=G) / jnp.maximum(cnt, 1.0)[:, None]
    max_p = jax.ops.segment_max(gat_out, batch, num_segments=G)
    max_p = jnp.where(jnp.isfinite(max_p), max_p, 0.0)
    pooled = jnp.concatenate([add_p, mean_p, max_p], axis=1)
    f = params['fusion']
    g1 = jax.nn.relu(_ln(_lin(pooled, f['W1'], f['b1']), f['ln_g'], f['ln_b']))
    g2 = jax.nn.relu(_lin(g1, f['W2'], f['b2']))
    q = params['pred']
    o = jax.nn.relu(_lin(g2, q['W1'], q['b1']))
    return _lin(o, q['W2'], q['b2'])


# speedup vs baseline: 2.0347x; 2.0347x over previous
import functools

import jax
import jax.numpy as jnp
from jax import lax
from jax.experimental import pallas as pl
from jax.experimental.pallas import tpu as pltpu
from jax.experimental.pallas import tpu_sc as plsc

N = 100000
E = 1600000
G = 64
IN_DIM = 24
GLOBAL_DIM = 10
H = 64
HEADS = 4
C = 16

NSC = 2
NTC = 16
SL = 4

K = 1024
GCH = K // 128
N_PAD = 100352
E_PAD = 98 * NTC * K
PER_TILE = E_PAD // NTC
N_CHUNKS = PER_TILE // K
ZROWS = N_PAD // NTC
ZB = 448

_mesh = plsc.VectorSubcoreMesh(core_axis_name="c", subcore_axis_name="s")


@functools.partial(
    pl.kernel,
    out_type=jax.ShapeDtypeStruct((SL, N_PAD, 16), jnp.float32),
    mesh=_mesh,
    scratch_types=[
        pltpu.VMEM((GCH, 128), jnp.int32),
        pltpu.VMEM((GCH, 128), jnp.int32),
        pltpu.VMEM((K, 16), jnp.float32),
        pltpu.VMEM((ZB, 16), jnp.float32),
        pltpu.VMEM_SHARED((N_PAD, 16), jnp.float32),
        pltpu.SemaphoreType.DMA,
    ],
    compiler_params=pltpu.CompilerParams(use_tc_tiling_on_sc=False),
    name="gin_agg",
)
def _gin_agg(hsl, src2, dst2, out, sidx, didx, rows, zbuf, acc, sem):
    c = lax.axis_index("c")
    s = lax.axis_index("s")

    @pl.loop(0, ZB)
    def _zero_zbuf(i):
        zbuf[i] = jnp.zeros((16,), jnp.float32)

    for si in range(2):
        sl_id = 2 * c + si
        hsl_s = hsl.at[sl_id]
        for j in range(ZROWS // ZB):
            pltpu.sync_copy(zbuf, acc.at[pl.ds(s * ZROWS + j * ZB, ZB)])
        plsc.subcore_barrier()

        @pl.loop(0, N_CHUNKS)
        def _chunk(ci):
            rb = s * (PER_TILE // 128) + ci * GCH
            pltpu.sync_copy(src2.at[pl.ds(rb, GCH)], sidx)
            pltpu.sync_copy(dst2.at[pl.ds(rb, GCH)], didx)
            descs = []
            for g in range(GCH):
                descs.append(pltpu.async_copy(
                    hsl_s.at[sidx.at[g]], rows.at[pl.ds(g * 128, 128)], sem))
            for d in descs:
                d.wait()
            for g in range(GCH):
                pltpu.sync_copy(rows.at[pl.ds(g * 128, 128)],
                                acc.at[didx.at[g]], add=True)

        plsc.subcore_barrier()
        pltpu.sync_copy(acc.at[pl.ds(s * ZROWS, ZROWS)],
                        out.at[sl_id, pl.ds(s * ZROWS, ZROWS)])
        plsc.subcore_barrier()


PTA = E_PAD // (NSC * NTC)
N_CH_A = PTA // K


@functools.partial(
    pl.kernel,
    out_type=(
        jax.ShapeDtypeStruct((E_PAD // 2, 16), jnp.float32),
        jax.ShapeDtypeStruct((NSC, N_PAD, 16), jnp.float32),
    ),
    mesh=_mesh,
    scratch_types=[
        pltpu.VMEM((GCH, 128), jnp.int32),
        pltpu.VMEM((GCH, 128), jnp.int32),
        pltpu.VMEM((K, 16), jnp.float32),
        pltpu.VMEM((K, 16), jnp.float32),
        pltpu.VMEM((K // 2, 16), jnp.float32),
        pltpu.VMEM((ZB, 16), jnp.float32),
        pltpu.VMEM_SHARED((N_PAD // 2 + 4096, 16), jnp.float32),
        pltpu.SemaphoreType.DMA,
    ],
    compiler_params=pltpu.CompilerParams(use_tc_tiling_on_sc=False),
    name="gat_att",
)
def _gat_att(asrc16, adst16, src2, dst2, ex4, den, sidx, didx, abuf, bbuf,
             pbuf, zbuf, acc, sem):
    c = lax.axis_index("c")
    s = lax.axis_index("s")
    w = s * NSC + c
    HN = N_PAD // 2

    @pl.loop(0, ZB)
    def _zz(i):
        zbuf[i] = jnp.zeros((16,), jnp.float32)

    iot = lax.iota(jnp.int32, 16)
    rot = (iot - 8) & 15
    unrot = (iot + 8) & 15
    msk = iot < 8

    for half in range(2):
        for j in range(ZROWS // ZB // 2):
            pltpu.sync_copy(zbuf, acc.at[pl.ds(s * (ZROWS // 2) + j * ZB, ZB)])
        pltpu.sync_copy(zbuf.at[pl.ds(0, 8)], acc.at[pl.ds(HN, 8)])
        plsc.subcore_barrier()

        @pl.loop(0, N_CH_A)
        def _chunk(ci):
            rb = w * (PTA // 128) + ci * GCH
            pltpu.sync_copy(dst2.at[pl.ds(rb, GCH)], didx)
            pltpu.sync_copy(src2.at[pl.ds(rb, GCH)], sidx)
            descs = []
            for g in range(GCH):
                descs.append(pltpu.async_copy(
                    asrc16.at[sidx.at[g]], abuf.at[pl.ds(g * 128, 128)],
                    sem))
                descs.append(pltpu.async_copy(
                    adst16.at[didx.at[g]], bbuf.at[pl.ds(g * 128, 128)],
                    sem))

            for d in descs:
                d.wait()

            for g in range(GCH):
                @pl.loop(0, 8)
                def _remap(j):
                    v = didx[g, pl.ds(j * 16, 16)] - half * HN
                    ok = (v >= 0) & (v < HN)
                    didx[g, pl.ds(j * 16, 16)] = jnp.where(
                        ok, v, HN + (v & 4095))

            @pl.loop(0, K, unroll=8)
            def _edge(j):
                v = abuf[j] + bbuf[j]
                v = jnp.where(v > 0, v, 0.2 * v)
                abuf[j] = jnp.exp(v)

            if half == 0:
                @pl.loop(0, K // 2, unroll=8)
                def _pair(q):
                    a = abuf[2 * q]
                    b = abuf[2 * q + 1]
                    pbuf[q] = jnp.where(msk, a, b[rot])

                pltpu.sync_copy(
                    pbuf, ex4.at[pl.ds((w * PTA + ci * K) // 2, K // 2)])

            for g in range(GCH):
                pltpu.sync_copy(abuf.at[pl.ds(g * 128, 128)],
                                acc.at[didx.at[g]], add=True)

        plsc.subcore_barrier()
        pltpu.sync_copy(
            acc.at[pl.ds(s * (ZROWS // 2), ZROWS // 2)],
            den.at[c, pl.ds(half * HN + s * (ZROWS // 2), ZROWS // 2)])
        plsc.subcore_barrier()


@functools.partial(
    pl.kernel,
    out_type=jax.ShapeDtypeStruct((HEADS, N_PAD, 16), jnp.float32),
    mesh=_mesh,
    scratch_types=[
        pltpu.VMEM((2, GCH, 128), jnp.int32),
        pltpu.VMEM((2, GCH, 128), jnp.int32),
        pltpu.VMEM((2, K, 16), jnp.float32),
        pltpu.VMEM((2, K // 2, 16), jnp.float32),
        pltpu.VMEM((ZB, 16), jnp.float32),
        pltpu.VMEM_SHARED((N_PAD // 2 + 4096, 16), jnp.float32),
        pltpu.SemaphoreType.DMA,
        pltpu.SemaphoreType.DMA,
    ],
    compiler_params=pltpu.CompilerParams(use_tc_tiling_on_sc=False),
    name="gat_agg",
)
def _gat_agg(xh_sl, ex4, src2, dst2, out, sidx, didx, rows, exv, zbuf,
             acc, gsem, ssem):
    c = lax.axis_index("c")
    s = lax.axis_index("s")
    HN = N_PAD // 2

    @pl.loop(0, ZB)
    def _zz(i):
        zbuf[i] = jnp.zeros((16,), jnp.float32)

    def fire(ci, b, xh_h):
        base = s * PER_TILE + ci * K
        rb = base // 128
        base2 = s * (PER_TILE // 2) + ci * (K // 2)
        pltpu.sync_copy(src2.at[pl.ds(rb, GCH)], sidx.at[b])
        pltpu.sync_copy(dst2.at[pl.ds(rb, GCH)], didx.at[b])
        pltpu.sync_copy(ex4.at[pl.ds(base2, K // 2)], exv.at[b])

        @pl.loop(0, GCH)
        def _fire(g):
            pltpu.async_copy(
                xh_h.at[sidx.at[b].at[g]],
                rows.at[b].at[pl.ds(g * 128, 128)], gsem)

    def process(b, hd, half, xh_h):
        pltpu.make_async_copy(
            xh_h.at[pl.ds(0, K)], rows.at[b], gsem).wait()
        for g in range(GCH):
            @pl.loop(0, 8)
            def _remap(j):
                v = didx[b, g, pl.ds(j * 16, 16)] - half * HN
                ok = (v >= 0) & (v < HN)
                didx[b, g, pl.ds(j * 16, 16)] = jnp.where(
                    ok, v, HN + (v & 4095))

        lane0 = jnp.full((16,), hd, jnp.int32)
        lane1 = jnp.full((16,), 8 + hd, jnp.int32)

        @pl.loop(0, K // 2, unroll=8)
        def _scale(q):
            v = exv[b, q]
            rows[b, 2 * q] = rows[b, 2 * q] * v[lane0]
            rows[b, 2 * q + 1] = rows[b, 2 * q + 1] * v[lane1]

        @pl.loop(0, GCH)
        def _scat(g):
            pltpu.async_copy(rows.at[b].at[pl.ds(g * 128, 128)],
                             acc.at[didx.at[b].at[g]], ssem, add=True)

    def drain_scatter(b):
        pltpu.make_async_copy(acc.at[pl.ds(0, K)], rows.at[b], ssem).wait()

    for si in range(2):
        hd = 2 * c + si
        xh_h = xh_sl.at[hd]
        for half in range(2):
            for j in range(ZROWS // ZB // 2):
                pltpu.sync_copy(
                    zbuf, acc.at[pl.ds(s * (ZROWS // 2) + j * ZB, ZB)])
            pltpu.sync_copy(zbuf.at[pl.ds(0, 8)], acc.at[pl.ds(HN, 8)])
            plsc.subcore_barrier()

            fire(0, 0, xh_h)

            @pl.loop(0, N_CHUNKS - 1)
            def _chunk(ci):
                b = ci % 2
                fire(ci + 1, 1 - b, xh_h)
                process(b, hd, half, xh_h)
                drain_scatter(b)

            process((N_CHUNKS - 1) % 2, hd, half, xh_h)
            drain_scatter((N_CHUNKS - 1) % 2)

            plsc.subcore_barrier()
            pltpu.sync_copy(
                acc.at[pl.ds(s * (ZROWS // 2), ZROWS // 2)],
                out.at[hd, pl.ds(half * HN + s * (ZROWS // 2), ZROWS // 2)])
            plsc.subcore_barrier()


def _segment_sum_edges(h, src2, dst2):
    hp = jnp.pad(h, ((0, N_PAD - N), (0, 0)))
    hsl = hp.reshape(N_PAD, SL, 16).transpose(1, 0, 2)
    agg_sl = _gin_agg(hsl, src2, dst2)
    return agg_sl.transpose(1, 0, 2).reshape(N_PAD, H)[:N]


def _ln(x, g, b):
    mu = jnp.mean(x, axis=-1, keepdims=True)
    var = jnp.var(x, axis=-1, keepdims=True)
    return (x - mu) / jnp.sqrt(var + 1e-5) * g + b


def _lin(x, W, b):
    return x @ W + b


def _gin_conv(x, src2, dst2, p):
    agg = _segment_sum_edges(x, src2, dst2)
    h = (1.0 + p['eps']) * x + agg
    h = _lin(h, p['W1'], p['b1'])
    h = h / jnp.sqrt(1.0 + 1e-5) * p['bn_g'] + p['bn_b']
    h = jax.nn.relu(h)
    return _lin(h, p['W2'], p['b2'])


def _gat_conv(x, src2, dst2, p):
    xh = x @ p['W']
    xhr = xh.reshape(N, HEADS, C)
    a_src = jnp.sum(xhr * p['att_src'], axis=-1)
    a_dst = jnp.sum(xhr * p['att_dst'], axis=-1)
    asrc16 = jnp.pad(a_src, ((0, N_PAD - N), (0, 16 - HEADS)))
    adst16 = jnp.pad(a_dst, ((0, N_PAD - N), (0, 16 - HEADS)))
    ex4, den2 = _gat_att(asrc16, adst16, src2, dst2)
    exl = jnp.exp(jax.nn.leaky_relu(a_src + a_dst, 0.2))
    den = den2[0, :N, :HEADS] + den2[1, :N, :HEADS] + exl
    xh_sl = jnp.pad(xh, ((0, N_PAD - N), (0, 0))).reshape(
        N_PAD, HEADS, C).transpose(1, 0, 2)
    agg_sl = _gat_agg(xh_sl, ex4, src2, dst2)
    agg = agg_sl.transpose(1, 0, 2)[:N]
    outr = (agg + exl[:, :, None] * xhr) / (den[:, :, None] + 1e-16)
    return outr.reshape(N, HEADS * C) + p['bias']


def kernel(x, edge_index, batch, global_feats, params):
    src = edge_index[0]
    dst = edge_index[1]
    pad = N + (jnp.arange(E_PAD - E, dtype=jnp.int32) % (N_PAD - N))
    src2 = jnp.concatenate([src, pad]).reshape(E_PAD // 128, 128)
    dst2 = jnp.concatenate([dst, pad]).reshape(E_PAD // 128, 128)

    h = jnp.concatenate([x, global_feats[batch]], axis=1)
    pr = params['proj']
    h = jax.nn.relu(_ln(_lin(h, pr['W'], pr['b']), pr['ln_g'], pr['ln_b']))
    for p in params['gin']:
        idn = h
        h = jax.nn.relu(_ln(_gin_conv(h, src2, dst2, p), p['ln_g'], p['ln_b'])) + idn
    gin_out = h
    for p in params['gat']:
        idn = h
        h = jax.nn.elu(_ln(_gat_conv(h, src2, dst2, p), p['ln_g'], p['ln_b'])) + idn
    gat_out = h
    add_p = jax.ops.segment_sum(gin_out, batch, num_segments=G)
    cnt = jax.ops.segment_sum(jnp.ones((N,), dtype=jnp.float32), batch, num_segments=G)
    mean_p = jax.ops.segment_sum(gat_out, batch, num_segments=G) / jnp.maximum(cnt, 1.0)[:, None]
    max_p = jax.ops.segment_max(gat_out, batch, num_segments=G)
    max_p = jnp.where(jnp.isfinite(max_p), max_p, 0.0)
    pooled = jnp.concatenate([add_p, mean_p, max_p], axis=1)
    f = params['fusion']
    g1 = jax.nn.relu(_ln(_lin(pooled, f['W1'], f['b1']), f['ln_g'], f['ln_b']))
    g2 = jax.nn.relu(_lin(g1, f['W2'], f['b2']))
    q = params['pred']
    o = jax.nn.relu(_lin(g2, q['W1'], q['b1']))
    return _lin(o, q['W2'], q['b2'])
